# Initial kernel scaffold; baseline (speedup 1.0000x reference)
import jax
import jax.numpy as jnp
from jax.experimental import pallas as pl


def kernel(x_patent, x_author, pn_g, pn_b, an_g, an_b, pl_W, pl_b, al_W, al_b, g1_W, g1_as, g1_ad, g1_b, g2_W, g2_as, g2_ad, g2_b, s1_Wl, s1_bl, s1_Wr, n1_g, n1_b, n2_g, n2_b, n3_g, n3_b, c1_W, c1_b, c2_W, c2_b, edge_index_cites, edge_index_author_of):
    raise NotImplementedError("write your pallas kernel here")


# speedup vs baseline: 37.9674x; 37.9674x over previous
import functools

import jax
import jax.numpy as jnp
from jax import lax
from jax.experimental import pallas as pl
from jax.experimental.pallas import tpu as pltpu
from jax.experimental.pallas import tpu_sc as plsc
from jax._src.pallas import mpmd

N = 10000
NP = 10240
D = 128
H = 4
C = 32
E = 320000
E2 = E + NP
RB = 640
GRID = NP // RB
NCORE = 2
NSUB = 16
NW = NCORE * NSUB
EPW = E2 // NW
CH = 48
NCHUNK = EPW // CH
RPT = NP // NSUB


def _ln(x, g, b):
    mu = jnp.mean(x, axis=-1, keepdims=True)
    var = jnp.mean((x - mu) ** 2, axis=-1, keepdims=True)
    return (x - mu) * lax.rsqrt(var + 1e-5) * g + b


def _gat_tail(xp, gW, scW):
    h = jnp.dot(xp, gW, preferred_element_type=jnp.float32)
    sc = jnp.dot(h, scW, preferred_element_type=jnp.float32)
    return h, sc


def _prep_body(x_ref, png_ref, pnb_ref, plW_ref, plb_ref, gW_ref, scW_ref,
               xp_ref, h_ref, sc_ref):
    ln = _ln(x_ref[...], png_ref[...], pnb_ref[...])
    xp = jnp.dot(ln, plW_ref[...], preferred_element_type=jnp.float32) + plb_ref[...]
    xp_ref[...] = xp
    h, sc = _gat_tail(xp, gW_ref[...], scW_ref[...])
    h_ref[...] = h
    sc_ref[...] = sc


def _merge(np_ref, dp_ref, xp_ref, gb_ref, ng_ref, nb_ref, rep16_ref):
    num = np_ref[0] + np_ref[1]
    den16 = dp_ref[0] + dp_ref[1]
    denx = jnp.dot(den16, rep16_ref[...], preferred_element_type=jnp.float32)
    p = num / (denx + 1e-16) + gb_ref[...]
    p = jnp.maximum(p, 0.0)
    p = _ln(p, ng_ref[...], nb_ref[...])
    return p + xp_ref[...]


def _merge_prep_body(np_ref, dp_ref, xp_ref, gb_ref, ng_ref, nb_ref,
                     rep16_ref, gW_ref, scW_ref,
                     xp2_ref, h_ref, sc_ref):
    xp2 = _merge(np_ref, dp_ref, xp_ref, gb_ref, ng_ref, nb_ref, rep16_ref)
    xp2_ref[...] = xp2
    h, sc = _gat_tail(xp2, gW_ref[...], scW_ref[...])
    h_ref[...] = h
    sc_ref[...] = sc


def _final_body(np_ref, dp_ref, xp_ref, gb_ref, ng_ref, nb_ref, rep16_ref,
                c1W_ref, c1b_ref, c2W_ref, c2b_ref, out_ref):
    xp3 = _merge(np_ref, dp_ref, xp_ref, gb_ref, ng_ref, nb_ref, rep16_ref)
    hc = jnp.dot(xp3, c1W_ref[...], preferred_element_type=jnp.float32) + c1b_ref[...]
    hc = jnp.maximum(hc, 0.0)
    out_ref[...] = jnp.dot(hc, c2W_ref[...],
                           preferred_element_type=jnp.float32) + c2b_ref[...]


def _vec(n):
    return pl.BlockSpec((n,), lambda b: (0,))


def _mat(m, n):
    return pl.BlockSpec((m, n), lambda b: (0, 0))


_XSPEC = pl.BlockSpec((RB, D), lambda b: (b, 0))
_P3 = pl.BlockSpec((2, RB, D), lambda b: (0, b, 0))
_P3D = pl.BlockSpec((2, RB, 16), lambda b: (0, b, 0))


def _f32(*shape):
    return jax.ShapeDtypeStruct(shape, jnp.float32)


def _tc_prep(x, png, pnb, plW, plb, gW, scW):
    return pl.pallas_call(
        _prep_body,
        grid=(GRID,),
        in_specs=[_XSPEC, _vec(D), _vec(D), _mat(D, D), _vec(D), _mat(D, D),
                  _mat(D, D)],
        out_specs=[_XSPEC, _XSPEC, _XSPEC],
        out_shape=[_f32(NP, D), _f32(NP, D), _f32(NP, D)],
    )(x, png, pnb, plW, plb, gW, scW)


def _tc_merge_prep(np1, dp1, xp, gb, ng, nb, rep16, gW, scW):
    return pl.pallas_call(
        _merge_prep_body,
        grid=(GRID,),
        in_specs=[_P3, _P3D, _XSPEC, _vec(D), _vec(D), _vec(D), _mat(16, D),
                  _mat(D, D), _mat(D, D)],
        out_specs=[_XSPEC, _XSPEC, _XSPEC],
        out_shape=[_f32(NP, D), _f32(NP, D), _f32(NP, D)],
    )(np1, dp1, xp, gb, ng, nb, rep16, gW, scW)


def _tc_final(np2, dp2, xp2, gb, ng, nb, rep16, c1W, c1b, c2W, c2b):
    return pl.pallas_call(
        _final_body,
        grid=(GRID,),
        in_specs=[_P3, _P3D, _XSPEC, _vec(D), _vec(D), _vec(D), _mat(16, D),
                  _mat(D, D // 2), _vec(D // 2), _mat(D // 2, 10), _vec(10)],
        out_specs=[pl.BlockSpec((RB, 10), lambda b: (b, 0))],
        out_shape=[_f32(NP, 10)],
    )(np2, dp2, xp2, gb, ng, nb, rep16, c1W, c1b, c2W, c2b)[0]


def _lane_shift(v, idx16):
    dn = lax.GatherDimensionNumbers(
        offset_dims=(), collapsed_slice_dims=(0,), start_index_map=(0,))
    return lax.gather(v, idx16[:, None], dn, (1,),
                      mode=lax.GatherScatterMode.PROMISE_IN_BOUNDS)


def _sc_gat(h, sc, src, dst):
    smesh = plsc.ScalarSubcoreMesh(axis_name="c", num_cores=NCORE)
    vmesh = plsc.VectorSubcoreMesh(core_axis_name="c", subcore_axis_name="s")

    def scs_fn(h_hbm, sc_hbm, src_hbm, dst_hbm, nout_hbm, dout_hbm,
               sidx, didx, didx8, hrows, erows, srows, drows,
               zrows, idxb, bounce, num_acc, den_acc, sem):
        pass

    def tec_fn(h_hbm, sc_hbm, src_hbm, dst_hbm, nout_hbm, dout_hbm,
               sidx, didx, didx8, hrows, erows, srows, drows,
               zrows, idxb, bounce, num_acc, den_acc, sem):
        cid = lax.axis_index("c")
        sid = lax.axis_index("s")
        wid = cid * NSUB + sid
        r0 = sid * RPT
        iota16 = lax.iota(jnp.int32, 16)
        mask01 = jnp.where(iota16 < H, 1.0, 0.0)
        shift4 = (iota16 + H) & 15
        zero16 = jnp.zeros((16,), jnp.float32)
        for r in range(16):
            for q in range(D // 16):
                zrows[r, pl.ds(q * 16, 16)] = zero16
        for k in range(RPT // 16):
            idxb[pl.ds(0, 16)] = r0 + k * 16 + iota16
            pltpu.sync_copy(zrows, num_acc.at[idxb])
        for k in range(RPT // 128):
            idxb[pl.ds(0, 16)] = sid * (RPT // 8) + k * 16 + iota16
            pltpu.sync_copy(zrows, den_acc.at[idxb])
        plsc.subcore_barrier()

        def chunk_body(i, carry):
            base = wid * EPW + i * CH
            pltpu.sync_copy(src_hbm.at[pl.ds(base, CH)], sidx)
            pltpu.sync_copy(dst_hbm.at[pl.ds(base, CH)], didx)
            for g in range(CH // 16):
                didx8[pl.ds(g * 16, 16)] = (
                    didx[pl.ds(g * 16, 16)] >> 3)
            pltpu.async_copy(h_hbm.at[sidx], hrows, sem).wait()
            pltpu.async_copy(sc_hbm.at[sidx], srows, sem).wait()
            pltpu.async_copy(sc_hbm.at[didx], drows, sem).wait()

            for g in range(CH // 16):
                d16f = (didx[pl.ds(g * 16, 16)] & 7).astype(jnp.float32)
                for l in range(16):
                    j = g * 16 + l
                    lf = jnp.full((16,), l, jnp.int32)
                    d7 = _lane_shift(d16f, lf)
                    sv = srows[j, pl.ds(0, 16)]
                    dv = drows[j, pl.ds(0, 16)]
                    z = sv + _lane_shift(dv, shift4)
                    e = jnp.exp(jnp.where(z > 0, z, 0.2 * z)) * mask01
                    for q in range(8):
                        mq = jnp.maximum(0.0, 1.0 - jnp.abs(d7 - float(q)))
                        erows[j, pl.ds(q * 16, 16)] = e * mq
                    eh = [_lane_shift(e, jnp.full((16,), hh, jnp.int32))
                          for hh in range(H)]
                    for q in range(D // 16):
                        seg = hrows[j, pl.ds(q * 16, 16)]
                        hrows[j, pl.ds(q * 16, 16)] = seg * eh[q // 2]
            pltpu.sync_copy(hrows, num_acc.at[didx], add=True)
            pltpu.sync_copy(erows, den_acc.at[didx8], add=True)
            return carry

        lax.fori_loop(0, NCHUNK, chunk_body, 0)
        plsc.subcore_barrier()
        for k in range(RPT // 16):
            idxb[pl.ds(0, 16)] = r0 + k * 16 + iota16
            pltpu.sync_copy(num_acc.at[idxb], bounce)
            pltpu.sync_copy(bounce, nout_hbm.at[cid, pl.ds(r0 + k * 16, 16)])
        for k in range(RPT // 128):
            r0d = sid * (RPT // 8) + k * 16
            idxb[pl.ds(0, 16)] = r0d + iota16
            pltpu.sync_copy(den_acc.at[idxb], bounce)
            pltpu.sync_copy(bounce, dout_hbm.at[cid, pl.ds(r0d, 16)])

    f = mpmd.mpmd_map(
        [(smesh, scs_fn), (vmesh, tec_fn)],
        out_types=[_f32(NCORE, NP, D), _f32(NCORE, NP // 8, D)],
        scratch_types=[
            pltpu.VMEM((CH,), jnp.int32) @ vmesh,
            pltpu.VMEM((CH,), jnp.int32) @ vmesh,
            pltpu.VMEM((CH,), jnp.int32) @ vmesh,
            pltpu.VMEM((CH, D), jnp.float32) @ vmesh,
            pltpu.VMEM((CH, D), jnp.float32) @ vmesh,
            pltpu.VMEM((CH, D), jnp.float32) @ vmesh,
            pltpu.VMEM((CH, D), jnp.float32) @ vmesh,
            pltpu.VMEM((16, D), jnp.float32) @ vmesh,
            pltpu.VMEM((16,), jnp.int32) @ vmesh,
            pltpu.VMEM((16, D), jnp.float32) @ vmesh,
            pltpu.VMEM_SHARED((NP, D), jnp.float32),
            pltpu.VMEM_SHARED((NP // 8, D), jnp.float32),
            pltpu.SemaphoreType.DMA @ vmesh,
        ],
    )
    return f(h, sc, src, dst)


def _build_scW(g_as, g_ad):
    scW = jnp.zeros((D, D), jnp.float32)
    for hh in range(H):
        scW = scW.at[hh * C:(hh + 1) * C, hh].set(g_as[hh])
        scW = scW.at[hh * C:(hh + 1) * C, H + hh].set(g_ad[hh])
    return scW


def kernel(x_patent, x_author, pn_g, pn_b, an_g, an_b, pl_W, pl_b, al_W, al_b,
           g1_W, g1_as, g1_ad, g1_b, g2_W, g2_as, g2_ad, g2_b, s1_Wl, s1_bl,
           s1_Wr, n1_g, n1_b, n2_g, n2_b, n3_g, n3_b, c1_W, c1_b, c2_W, c2_b,
           edge_index_cites, edge_index_author_of):
    loops = jnp.arange(NP, dtype=jnp.int32)
    src = jnp.concatenate([edge_index_cites[0], loops])
    dst = jnp.concatenate([edge_index_cites[1], loops])
    scW1 = _build_scW(g1_as, g1_ad)
    scW2 = _build_scW(g2_as, g2_ad)
    rep16 = jnp.concatenate(
        [jnp.repeat(jnp.eye(H, dtype=jnp.float32), C, axis=1),
         jnp.zeros((12, D), jnp.float32)], 0)
    x_pad = jnp.zeros((NP, D), jnp.float32).at[:N].set(x_patent)

    xp, h1, sc1 = _tc_prep(x_pad, pn_g, pn_b, pl_W, pl_b, g1_W, scW1)
    np1, dp1 = _sc_gat(h1, sc1, src, dst)
    xp2, h2, sc2 = _tc_merge_prep(
        np1, dp1.reshape(NCORE, NP, 16), xp, g1_b, n1_g, n1_b, rep16,
        g2_W, scW2)
    np2, dp2 = _sc_gat(h2, sc2, src, dst)
    out = _tc_final(np2, dp2.reshape(NCORE, NP, 16), xp2, g2_b, n3_g, n3_b,
                    rep16, c1_W, c1_b, c2_W, c2_b)
    return out[:N]


# speedup vs baseline: 57.1308x; 1.5047x over previous
import functools

import jax
import jax.numpy as jnp
from jax import lax
from jax.experimental import pallas as pl
from jax.experimental.pallas import tpu as pltpu
from jax.experimental.pallas import tpu_sc as plsc
from jax._src.pallas import mpmd

N = 10000
NP = 10240
D = 128
H = 4
C = 32
E = 320000
E2 = E + NP
RB = 640
GRID = NP // RB
NCORE = 2
NSUB = 16
NW = NCORE * NSUB
EPW = E2 // NW
CH = 48
NCHUNK = EPW // CH
RPT = NP // NSUB


def _ln(x, g, b):
    mu = jnp.mean(x, axis=-1, keepdims=True)
    var = jnp.mean((x - mu) ** 2, axis=-1, keepdims=True)
    return (x - mu) * lax.rsqrt(var + 1e-5) * g + b


def _gat_tail(xp, gW, scW):
    h = jnp.dot(xp, gW, preferred_element_type=jnp.float32)
    sc = jnp.dot(h, scW, preferred_element_type=jnp.float32)
    return h, sc


def _prep_body(x_ref, png_ref, pnb_ref, plW_ref, plb_ref, gW_ref, scW_ref,
               xp_ref, h_ref, sc_ref):
    ln = _ln(x_ref[...], png_ref[...], pnb_ref[...])
    xp = jnp.dot(ln, plW_ref[...], preferred_element_type=jnp.float32) + plb_ref[...]
    xp_ref[...] = xp
    h, sc = _gat_tail(xp, gW_ref[...], scW_ref[...])
    h_ref[...] = h
    sc_ref[...] = sc


def _merge(np_ref, dp_ref, xp_ref, gb_ref, ng_ref, nb_ref, rep16_ref):
    num = np_ref[0] + np_ref[1]
    den16 = dp_ref[0] + dp_ref[1]
    denx = jnp.dot(den16, rep16_ref[...], preferred_element_type=jnp.float32)
    p = num / (denx + 1e-16) + gb_ref[...]
    p = jnp.maximum(p, 0.0)
    p = _ln(p, ng_ref[...], nb_ref[...])
    return p + xp_ref[...]


def _merge_prep_body(np_ref, dp_ref, xp_ref, gb_ref, ng_ref, nb_ref,
                     rep16_ref, gW_ref, scW_ref,
                     xp2_ref, h_ref, sc_ref):
    xp2 = _merge(np_ref, dp_ref, xp_ref, gb_ref, ng_ref, nb_ref, rep16_ref)
    xp2_ref[...] = xp2
    h, sc = _gat_tail(xp2, gW_ref[...], scW_ref[...])
    h_ref[...] = h
    sc_ref[...] = sc


def _final_body(np_ref, dp_ref, xp_ref, gb_ref, ng_ref, nb_ref, rep16_ref,
                c1W_ref, c1b_ref, c2W_ref, c2b_ref, out_ref):
    xp3 = _merge(np_ref, dp_ref, xp_ref, gb_ref, ng_ref, nb_ref, rep16_ref)
    hc = jnp.dot(xp3, c1W_ref[...], preferred_element_type=jnp.float32) + c1b_ref[...]
    hc = jnp.maximum(hc, 0.0)
    out_ref[...] = jnp.dot(hc, c2W_ref[...],
                           preferred_element_type=jnp.float32) + c2b_ref[...]


def _vec(n):
    return pl.BlockSpec((n,), lambda b: (0,))


def _mat(m, n):
    return pl.BlockSpec((m, n), lambda b: (0, 0))


_XSPEC = pl.BlockSpec((RB, D), lambda b: (b, 0))
_P3 = pl.BlockSpec((2, RB, D), lambda b: (0, b, 0))
_P3D = pl.BlockSpec((2, RB, 16), lambda b: (0, b, 0))


def _f32(*shape):
    return jax.ShapeDtypeStruct(shape, jnp.float32)


def _tc_prep(x, png, pnb, plW, plb, gW, scW):
    return pl.pallas_call(
        _prep_body,
        grid=(GRID,),
        in_specs=[_XSPEC, _vec(D), _vec(D), _mat(D, D), _vec(D), _mat(D, D),
                  _mat(D, D)],
        out_specs=[_XSPEC, _XSPEC, _XSPEC],
        out_shape=[_f32(NP, D), _f32(NP, D), _f32(NP, D)],
    )(x, png, pnb, plW, plb, gW, scW)


def _tc_merge_prep(np1, dp1, xp, gb, ng, nb, rep16, gW, scW):
    return pl.pallas_call(
        _merge_prep_body,
        grid=(GRID,),
        in_specs=[_P3, _P3D, _XSPEC, _vec(D), _vec(D), _vec(D), _mat(16, D),
                  _mat(D, D), _mat(D, D)],
        out_specs=[_XSPEC, _XSPEC, _XSPEC],
        out_shape=[_f32(NP, D), _f32(NP, D), _f32(NP, D)],
    )(np1, dp1, xp, gb, ng, nb, rep16, gW, scW)


def _tc_final(np2, dp2, xp2, gb, ng, nb, rep16, c1W, c1b, c2W, c2b):
    return pl.pallas_call(
        _final_body,
        grid=(GRID,),
        in_specs=[_P3, _P3D, _XSPEC, _vec(D), _vec(D), _vec(D), _mat(16, D),
                  _mat(D, D // 2), _vec(D // 2), _mat(D // 2, 10), _vec(10)],
        out_specs=[pl.BlockSpec((RB, 10), lambda b: (b, 0))],
        out_shape=[_f32(NP, 10)],
    )(np2, dp2, xp2, gb, ng, nb, rep16, c1W, c1b, c2W, c2b)[0]


def _lane_shift(v, idx16):
    dn = lax.GatherDimensionNumbers(
        offset_dims=(), collapsed_slice_dims=(0,), start_index_map=(0,))
    return lax.gather(v, idx16[:, None], dn, (1,),
                      mode=lax.GatherScatterMode.PROMISE_IN_BOUNDS)


def _sc_gat(h, sc, src, dst):
    smesh = plsc.ScalarSubcoreMesh(axis_name="c", num_cores=NCORE)
    vmesh = plsc.VectorSubcoreMesh(core_axis_name="c", subcore_axis_name="s")

    def scs_fn(h_hbm, sc_hbm, src_hbm, dst_hbm, nout_hbm, dout_hbm,
               sidx, didx, didx8, hrows, erows, srows, drows,
               zrows, idxb, bounce, num_acc, den_acc, sem):
        pass

    def tec_fn(h_hbm, sc_hbm, src_hbm, dst_hbm, nout_hbm, dout_hbm,
               sidx, didx, didx8, hrows, erows, srows, drows,
               zrows, idxb, bounce, num_acc, den_acc, sem):
        cid = lax.axis_index("c")
        sid = lax.axis_index("s")
        wid = cid * NSUB + sid
        r0 = sid * RPT
        iota16 = lax.iota(jnp.int32, 16)
        mask01 = jnp.where(iota16 < H, 1.0, 0.0)
        shift4 = (iota16 + H) & 15
        zero16 = jnp.zeros((16,), jnp.float32)
        for r in range(16):
            for q in range(D // 16):
                zrows[r, pl.ds(q * 16, 16)] = zero16
        for k in range(RPT // 16):
            idxb[pl.ds(0, 16)] = r0 + k * 16 + iota16
            pltpu.sync_copy(zrows, num_acc.at[idxb])
        for k in range(RPT // 128):
            idxb[pl.ds(0, 16)] = sid * (RPT // 8) + k * 16 + iota16
            pltpu.sync_copy(zrows, den_acc.at[idxb])
        plsc.subcore_barrier()

        def chunk_body(i, carry):
            base = wid * EPW + i * CH
            c1 = pltpu.async_copy(src_hbm.at[pl.ds(base, CH)], sidx, sem)
            c2 = pltpu.async_copy(dst_hbm.at[pl.ds(base, CH)], didx, sem)
            c1.wait()
            c2.wait()
            g1 = pltpu.async_copy(h_hbm.at[sidx], hrows, sem)
            g2 = pltpu.async_copy(sc_hbm.at[sidx], srows, sem)
            g3 = pltpu.async_copy(sc_hbm.at[didx], drows, sem)
            for g in range(CH // 16):
                didx8[pl.ds(g * 16, 16)] = (
                    didx[pl.ds(g * 16, 16)] >> 3)
            g1.wait()
            g2.wait()
            g3.wait()

            for g in range(CH // 16):
                d16f = (didx[pl.ds(g * 16, 16)] & 7).astype(jnp.float32)
                for l in range(16):
                    j = g * 16 + l
                    lf = jnp.full((16,), l, jnp.int32)
                    d7 = _lane_shift(d16f, lf)
                    sv = srows[j, pl.ds(0, 16)]
                    dv = drows[j, pl.ds(0, 16)]
                    z = sv + _lane_shift(dv, shift4)
                    e = jnp.exp(jnp.where(z > 0, z, 0.2 * z)) * mask01
                    for q in range(8):
                        mq = jnp.maximum(0.0, 1.0 - jnp.abs(d7 - float(q)))
                        erows[j, pl.ds(q * 16, 16)] = e * mq
                    eh = [_lane_shift(e, jnp.full((16,), hh, jnp.int32))
                          for hh in range(H)]
                    for q in range(D // 16):
                        seg = hrows[j, pl.ds(q * 16, 16)]
                        hrows[j, pl.ds(q * 16, 16)] = seg * eh[q // 2]
            s1 = pltpu.async_copy(hrows, num_acc.at[didx], sem, add=True)
            s2 = pltpu.async_copy(erows, den_acc.at[didx8], sem, add=True)
            s1.wait()
            s2.wait()
            return carry

        lax.fori_loop(0, NCHUNK, chunk_body, 0)
        plsc.subcore_barrier()
        for k in range(RPT // 16):
            idxb[pl.ds(0, 16)] = r0 + k * 16 + iota16
            pltpu.sync_copy(num_acc.at[idxb], bounce)
            pltpu.sync_copy(bounce, nout_hbm.at[cid, pl.ds(r0 + k * 16, 16)])
        for k in range(RPT // 128):
            r0d = sid * (RPT // 8) + k * 16
            idxb[pl.ds(0, 16)] = r0d + iota16
            pltpu.sync_copy(den_acc.at[idxb], bounce)
            pltpu.sync_copy(bounce, dout_hbm.at[cid, pl.ds(r0d, 16)])

    f = mpmd.mpmd_map(
        [(smesh, scs_fn), (vmesh, tec_fn)],
        out_types=[_f32(NCORE, NP, D), _f32(NCORE, NP // 8, D)],
        scratch_types=[
            pltpu.VMEM((CH,), jnp.int32) @ vmesh,
            pltpu.VMEM((CH,), jnp.int32) @ vmesh,
            pltpu.VMEM((CH,), jnp.int32) @ vmesh,
            pltpu.VMEM((CH, D), jnp.float32) @ vmesh,
            pltpu.VMEM((CH, D), jnp.float32) @ vmesh,
            pltpu.VMEM((CH, D), jnp.float32) @ vmesh,
            pltpu.VMEM((CH, D), jnp.float32) @ vmesh,
            pltpu.VMEM((16, D), jnp.float32) @ vmesh,
            pltpu.VMEM((16,), jnp.int32) @ vmesh,
            pltpu.VMEM((16, D), jnp.float32) @ vmesh,
            pltpu.VMEM_SHARED((NP, D), jnp.float32),
            pltpu.VMEM_SHARED((NP // 8, D), jnp.float32),
            pltpu.SemaphoreType.DMA @ vmesh,
        ],
    )
    return f(h, sc, src, dst)


def _build_scW(g_as, g_ad):
    scW = jnp.zeros((D, D), jnp.float32)
    for hh in range(H):
        scW = scW.at[hh * C:(hh + 1) * C, hh].set(g_as[hh])
        scW = scW.at[hh * C:(hh + 1) * C, H + hh].set(g_ad[hh])
    return scW


def kernel(x_patent, x_author, pn_g, pn_b, an_g, an_b, pl_W, pl_b, al_W, al_b,
           g1_W, g1_as, g1_ad, g1_b, g2_W, g2_as, g2_ad, g2_b, s1_Wl, s1_bl,
           s1_Wr, n1_g, n1_b, n2_g, n2_b, n3_g, n3_b, c1_W, c1_b, c2_W, c2_b,
           edge_index_cites, edge_index_author_of):
    loops = jnp.arange(NP, dtype=jnp.int32)
    src = jnp.concatenate([edge_index_cites[0], loops])
    dst = jnp.concatenate([edge_index_cites[1], loops])
    scW1 = _build_scW(g1_as, g1_ad)
    scW2 = _build_scW(g2_as, g2_ad)
    rep16 = jnp.concatenate(
        [jnp.repeat(jnp.eye(H, dtype=jnp.float32), C, axis=1),
         jnp.zeros((12, D), jnp.float32)], 0)
    x_pad = jnp.zeros((NP, D), jnp.float32).at[:N].set(x_patent)

    xp, h1, sc1 = _tc_prep(x_pad, pn_g, pn_b, pl_W, pl_b, g1_W, scW1)
    np1, dp1 = _sc_gat(h1, sc1, src, dst)
    xp2, h2, sc2 = _tc_merge_prep(
        np1, dp1.reshape(NCORE, NP, 16), xp, g1_b, n1_g, n1_b, rep16,
        g2_W, scW2)
    np2, dp2 = _sc_gat(h2, sc2, src, dst)
    out = _tc_final(np2, dp2.reshape(NCORE, NP, 16), xp2, g2_b, n3_g, n3_b,
                    rep16, c1_W, c1_b, c2_W, c2_b)
    return out[:N]
